# baseline (speedup 1.0000x reference)
import jax
import jax.numpy as jnp
from jax import lax
from jax.experimental import pallas as pl
from jax.experimental.pallas import tpu as pltpu

_TT = 1024
_GT = 16
_RR = 512


def _relayout_body(cb_ref, out_ref):
    out_ref[...] = cb_ref[...].reshape(out_ref.shape)


def _gather_body(idx_ref, cb_ref, out_ref, buf_ref):
    g = pl.program_id(0)
    par = lax.rem(g, 2)
    wbase = par * (_TT * 8)
    rbase = (_TT * 8) - wbase

    def it(j, c):
        base = j * _GT
        for k in range(_GT):
            row8 = idx_ref[0, 0, base + k]
            dst = pl.multiple_of(wbase + 8 * (base + k), 8)
            buf_ref[pl.ds(dst, 8), :] = cb_ref[pl.ds(row8, 8), :]
        s = lax.rem(j, 8)
        tb = (j // 8) * 128
        slab = buf_ref[pl.Slice(rbase + 8 * tb + s, 128, 8), :]
        out_ref[0, pl.ds(128 * s, 128), pl.ds(tb, 128)] = slab.T
        return c

    lax.fori_loop(0, _TT // _GT, it, 0, unroll=2)


def kernel(indices, codebook):
    B, T = indices.shape
    V, D = codebook.shape
    NT = T // _TT
    G = B * NT + 1
    idx = jnp.clip(indices.astype(jnp.int32), 0, V - 1) * 8
    idx = idx.reshape(B * NT, 1, _TT)

    cb2 = pl.pallas_call(
        _relayout_body,
        grid=(V // _RR,),
        in_specs=[pl.BlockSpec((_RR, D), lambda r: (r, 0))],
        out_specs=pl.BlockSpec((_RR * 8, D // 8), lambda r: (r, 0)),
        out_shape=jax.ShapeDtypeStruct((V * 8, D // 8), jnp.float32),
    )(codebook)

    def idx_map(g):
        return (jnp.minimum(g, B * NT - 1), 0, 0)

    def out_map(g):
        h = jnp.maximum(g - 1, 0)
        return (h // NT, 0, h % NT)

    out = pl.pallas_call(
        _gather_body,
        grid=(G,),
        in_specs=[
            pl.BlockSpec((1, 1, _TT), idx_map, memory_space=pltpu.SMEM),
            pl.BlockSpec((V * 8, D // 8), lambda g: (0, 0)),
        ],
        out_specs=pl.BlockSpec((1, D, _TT), out_map),
        out_shape=jax.ShapeDtypeStruct((B, D, T), jnp.float32),
        scratch_shapes=[pltpu.VMEM((2 * _TT * 8, D // 8), jnp.float32)],
    )(idx, cb2)
    return out


# speedup vs baseline: 1.9523x; 1.9523x over previous
import jax
import jax.numpy as jnp
from jax import lax
from jax.experimental import pallas as pl
from jax.experimental.pallas import tpu as pltpu

_TT = 2048
_RR = 512


def _relayout_body(cb_ref, out_ref):
    out_ref[...] = cb_ref[...].reshape(out_ref.shape)


def _gather_body(idx_ref, cb_ref, out_ref, scratch_ref):
    def tok(i, carry):
        row8 = idx_ref[0, 0, i]
        scratch_ref[pl.ds(8 * i, 8), :] = cb_ref[pl.ds(row8, 8), :]
        return carry

    lax.fori_loop(0, _TT, tok, 0, unroll=32)
    for s in range(8):
        slab = scratch_ref[pl.Slice(s, _TT, 8), :]
        out_ref[0, pl.ds(128 * s, 128), :] = slab.T


def kernel(indices, codebook):
    B, T = indices.shape
    V, D = codebook.shape
    NT = T // _TT
    idx = jnp.clip(indices.astype(jnp.int32), 0, V - 1) * 8
    idx = idx.reshape(B * NT, 1, _TT)

    cb2 = pl.pallas_call(
        _relayout_body,
        grid=(V // _RR,),
        in_specs=[pl.BlockSpec((_RR, D), lambda r: (r, 0))],
        out_specs=pl.BlockSpec((_RR * 8, D // 8), lambda r: (r, 0)),
        out_shape=jax.ShapeDtypeStruct((V * 8, D // 8), jnp.float32),
    )(codebook)

    out = pl.pallas_call(
        _gather_body,
        grid=(B, NT),
        in_specs=[
            pl.BlockSpec((1, 1, _TT), lambda b, t: (b * NT + t, 0, 0),
                         memory_space=pltpu.SMEM),
            pl.BlockSpec((V * 8, D // 8), lambda b, t: (0, 0)),
        ],
        out_specs=pl.BlockSpec((1, D, _TT), lambda b, t: (b, 0, t)),
        out_shape=jax.ShapeDtypeStruct((B, D, T), jnp.float32),
        scratch_shapes=[pltpu.VMEM((_TT * 8, D // 8), jnp.float32)],
    )(idx, cb2)
    return out


# speedup vs baseline: 2.0030x; 1.0259x over previous
import jax
import jax.numpy as jnp
from jax import lax
from jax.experimental import pallas as pl
from jax.experimental.pallas import tpu as pltpu

_TT = 2048
_RR = 512


def _relayout_body(cb_ref, out_ref):
    out_ref[...] = cb_ref[...].reshape(out_ref.shape)


def _gather_body(idx_ref, cb_ref, out_ref, scratch_ref):
    def tok(i, carry):
        row8 = idx_ref[0, 0, i]
        scratch_ref[pl.ds(8 * i, 8), :] = cb_ref[pl.ds(row8, 8), :]
        return carry

    lax.fori_loop(0, _TT, tok, 0, unroll=64)
    for s in range(8):
        slab = scratch_ref[pl.Slice(s, _TT, 8), :]
        out_ref[0, pl.ds(128 * s, 128), :] = slab.T


def kernel(indices, codebook):
    B, T = indices.shape
    V, D = codebook.shape
    NT = T // _TT
    idx = jnp.clip(indices.astype(jnp.int32), 0, V - 1) * 8
    idx = idx.reshape(B * NT, 1, _TT)

    cb2 = pl.pallas_call(
        _relayout_body,
        grid=(V // _RR,),
        in_specs=[pl.BlockSpec((_RR, D), lambda r: (r, 0))],
        out_specs=pl.BlockSpec((_RR * 8, D // 8), lambda r: (r, 0)),
        out_shape=jax.ShapeDtypeStruct((V * 8, D // 8), jnp.float32),
    )(codebook)

    out = pl.pallas_call(
        _gather_body,
        grid=(B, NT),
        in_specs=[
            pl.BlockSpec((1, 1, _TT), lambda b, t: (b * NT + t, 0, 0),
                         memory_space=pltpu.SMEM),
            pl.BlockSpec((V * 8, D // 8), lambda b, t: (0, 0)),
        ],
        out_specs=pl.BlockSpec((1, D, _TT), lambda b, t: (b, 0, t)),
        out_shape=jax.ShapeDtypeStruct((B, D, T), jnp.float32),
        scratch_shapes=[pltpu.VMEM((_TT * 8, D // 8), jnp.float32)],
    )(idx, cb2)
    return out
